# baseline (speedup 1.0000x reference)
import functools

import jax
import jax.numpy as jnp
from jax import lax
from jax.experimental import pallas as pl
from jax.experimental.pallas import tpu as pltpu
from jax.experimental.pallas import tpu_sc as plsc

N_ROWS = 2097152
F = N_ROWS * 8
NC = 2
NS = 16
NW = NC * NS
ROWS_PER_W = N_ROWS // NW
ROWS_CHUNK = 256
N_CHUNKS = ROWS_PER_W // ROWS_CHUNK
GROUPS = ROWS_CHUNK // 16

MAGIC = 12582912.0


def _quantize_group(xs):
    f_ = [(xs[c] + MAGIC) - MAGIC for c in range(8)]
    d_ = [xs[c] - f_[c] for c in range(8)]
    a_ = [jnp.abs(d_[c]) for c in range(8)]
    neg = [d_[c] < 0.0 for c in range(8)]
    sgn = [jnp.where(neg[c], -1.0, 1.0) for c in range(8)]
    ind = [jnp.where(neg[c], 1.0, 0.0) for c in range(8)]

    def tree(op, vs):
        t0 = op(vs[0], vs[1]); t1 = op(vs[2], vs[3])
        t2 = op(vs[4], vs[5]); t3 = op(vs[6], vs[7])
        return op(op(t0, t1), op(t2, t3))

    m_a = tree(jnp.maximum, a_)
    m_n = tree(jnp.minimum, a_)
    sum_f = tree(jnp.add, f_)
    sum_i = tree(jnp.add, ind)
    sum_a = tree(jnp.add, a_)
    sq_a = tree(jnp.add, [d_[c] * d_[c] for c in range(8)])
    sq_b = (sq_a + 2.0) - sum_a

    k_a = jnp.zeros((16,), jnp.int32)
    k_b = jnp.zeros((16,), jnp.int32)
    fix_a = jnp.zeros((16,), jnp.float32)
    s_b = jnp.zeros((16,), jnp.float32)
    for c in range(7, -1, -1):
        ck_a = a_[c] == m_a
        ck_b = a_[c] == m_n
        k_a = jnp.where(ck_a, c, k_a)
        k_b = jnp.where(ck_b, c, k_b)
        fix_a = jnp.where(ck_a, sgn[c], fix_a)
        s_b = jnp.where(ck_b, sgn[c], s_b)
    fix_b = -s_b

    odd_a = lax.rem(sum_f, 2.0) != 0.0
    sum_fb = sum_f - sum_i
    odd_b = lax.rem(sum_fb, 2.0) != 0.0

    sq_ap = sq_a + jnp.where(odd_a, 1.0 - 2.0 * m_a, 0.0)
    sq_bp = sq_b + jnp.where(odd_b, 2.0 * m_n, 0.0)
    win_b = sq_bp < sq_ap

    k_w = jnp.where(win_b, k_b, k_a)
    val_a = jnp.where(odd_a, fix_a, 0.0)
    val_b = jnp.where(odd_b, fix_b, 0.0)
    val_w = jnp.where(win_b, val_b, val_a)

    ys = []
    for c in range(8):
        yc = f_[c] + jnp.where(win_b, 0.5 - ind[c], 0.0)
        yc = yc + jnp.where(k_w == c, val_w, 0.0)
        ys.append(yc)
    return ys


def _sc_body(x_hbm, out_hbm, buf_in, buf_out):
    cid = lax.axis_index("c")
    sid = lax.axis_index("s")
    wid = sid * NC + cid
    base_rows = wid * ROWS_PER_W
    iota = lax.iota(jnp.int32, 16)
    cols = [jnp.full((16,), c, jnp.int32) for c in range(8)]

    def chunk_body(i, carry):
        r0 = base_rows + i * ROWS_CHUNK
        pltpu.sync_copy(x_hbm.at[pl.ds(r0, ROWS_CHUNK)], buf_in)

        def group_body(g, c2):
            rows = iota + g * 16
            xs = [plsc.load_gather(buf_in, [rows, cols[c]]) for c in range(8)]
            ys = _quantize_group(xs)
            for c in range(8):
                plsc.store_scatter(buf_out, [rows, cols[c]], ys[c])
            return c2

        lax.fori_loop(0, GROUPS, group_body, 0)
        pltpu.sync_copy(buf_out, out_hbm.at[pl.ds(r0, ROWS_CHUNK)])
        return carry

    lax.fori_loop(0, N_CHUNKS, chunk_body, 0)


@jax.jit
def _e8_quantize(x):
    run = functools.partial(
        pl.kernel,
        out_type=jax.ShapeDtypeStruct((N_ROWS, 8), jnp.float32),
        mesh=plsc.VectorSubcoreMesh(core_axis_name="c", subcore_axis_name="s"),
        scratch_types=[
            pltpu.VMEM((ROWS_CHUNK, 8), jnp.float32),
            pltpu.VMEM((ROWS_CHUNK, 8), jnp.float32),
        ],
        compiler_params=pltpu.CompilerParams(
            needs_layout_passes=False, use_tc_tiling_on_sc=True),
    )
    return run(_sc_body)(x)


def kernel(x):
    return _e8_quantize(x)


# speedup vs baseline: 7.9378x; 7.9378x over previous
import functools

import jax
import jax.numpy as jnp
from jax import lax
from jax.experimental import pallas as pl
from jax.experimental.pallas import tpu as pltpu
from jax.experimental.pallas import tpu_sc as plsc

N_ROWS = 2097152
NC = 2
NS = 16
NW = NC * NS
COLS_PER_W = N_ROWS // NW
COLS_CHUNK = 2048
N_CHUNKS = COLS_PER_W // COLS_CHUNK
GROUPS = COLS_CHUNK // 16

MAGIC = 12582912.0


def _quantize_group(xs):
    f_ = [(xs[c] + MAGIC) - MAGIC for c in range(8)]
    d_ = [xs[c] - f_[c] for c in range(8)]
    a_ = [jnp.abs(d_[c]) for c in range(8)]
    neg = [d_[c] < 0.0 for c in range(8)]
    sgn = [jnp.where(neg[c], -1.0, 1.0) for c in range(8)]
    ind = [jnp.where(neg[c], 1.0, 0.0) for c in range(8)]

    def tree(op, vs):
        t0 = op(vs[0], vs[1]); t1 = op(vs[2], vs[3])
        t2 = op(vs[4], vs[5]); t3 = op(vs[6], vs[7])
        return op(op(t0, t1), op(t2, t3))

    m_a = tree(jnp.maximum, a_)
    m_n = tree(jnp.minimum, a_)
    sum_f = tree(jnp.add, f_)
    sum_i = tree(jnp.add, ind)
    sum_a = tree(jnp.add, a_)
    sq_a = tree(jnp.add, [d_[c] * d_[c] for c in range(8)])
    sq_b = (sq_a + 2.0) - sum_a

    k_a = jnp.zeros((16,), jnp.int32)
    k_b = jnp.zeros((16,), jnp.int32)
    fix_a = jnp.zeros((16,), jnp.float32)
    s_b = jnp.zeros((16,), jnp.float32)
    for c in range(7, -1, -1):
        ck_a = a_[c] == m_a
        ck_b = a_[c] == m_n
        k_a = jnp.where(ck_a, c, k_a)
        k_b = jnp.where(ck_b, c, k_b)
        fix_a = jnp.where(ck_a, sgn[c], fix_a)
        s_b = jnp.where(ck_b, sgn[c], s_b)
    fix_b = -s_b

    odd_a = lax.rem(sum_f, 2.0) != 0.0
    sum_fb = sum_f - sum_i
    odd_b = lax.rem(sum_fb, 2.0) != 0.0

    sq_ap = sq_a + jnp.where(odd_a, 1.0 - 2.0 * m_a, 0.0)
    sq_bp = sq_b + jnp.where(odd_b, 2.0 * m_n, 0.0)
    win_b = sq_bp < sq_ap

    k_w = jnp.where(win_b, k_b, k_a)
    val_a = jnp.where(odd_a, fix_a, 0.0)
    val_b = jnp.where(odd_b, fix_b, 0.0)
    val_w = jnp.where(win_b, val_b, val_a)

    ys = []
    for c in range(8):
        yc = f_[c] + jnp.where(win_b, 0.5 - ind[c], 0.0)
        yc = yc + jnp.where(k_w == c, val_w, 0.0)
        ys.append(yc)
    return ys


def _sc_body(xt_hbm, out_hbm, buf_in, buf_out):
    cid = lax.axis_index("c")
    sid = lax.axis_index("s")
    wid = sid * NC + cid
    base_col = wid * COLS_PER_W

    def chunk_body(i, carry):
        c0 = base_col + i * COLS_CHUNK
        pltpu.sync_copy(xt_hbm.at[:, pl.ds(c0, COLS_CHUNK)], buf_in)

        def group_body(g, c2):
            o = g * 16
            xs = [buf_in[c, pl.ds(o, 16)] for c in range(8)]
            ys = _quantize_group(xs)
            for c in range(8):
                buf_out[c, pl.ds(o, 16)] = ys[c]
            return c2

        lax.fori_loop(0, GROUPS, group_body, 0)
        pltpu.sync_copy(buf_out, out_hbm.at[:, pl.ds(c0, COLS_CHUNK)])
        return carry

    lax.fori_loop(0, N_CHUNKS, chunk_body, 0)


@jax.jit
def _e8_quantize_t(xt):
    run = functools.partial(
        pl.kernel,
        out_type=jax.ShapeDtypeStruct((8, N_ROWS), jnp.float32),
        mesh=plsc.VectorSubcoreMesh(core_axis_name="c", subcore_axis_name="s"),
        scratch_types=[
            pltpu.VMEM((8, COLS_CHUNK), jnp.float32),
            pltpu.VMEM((8, COLS_CHUNK), jnp.float32),
        ],
        compiler_params=pltpu.CompilerParams(use_tc_tiling_on_sc=True),
    )
    return run(_sc_body)(xt)


def kernel(x):
    return _e8_quantize_t(x.T).T


# speedup vs baseline: 12.7832x; 1.6104x over previous
import functools

import jax
import jax.numpy as jnp
from jax import lax
from jax.experimental import pallas as pl
from jax.experimental.pallas import tpu as pltpu
from jax.experimental.pallas import tpu_sc as plsc

N_ROWS = 2097152
NC = 2
NS = 16
NW = NC * NS
COLS_PER_W = N_ROWS // NW
COLS_CHUNK = 2048
N_CHUNKS = COLS_PER_W // COLS_CHUNK
GROUPS = COLS_CHUNK // 16

MAGIC = 12582912.0


def _quantize_group(xs):
    f_ = [(xs[c] + MAGIC) - MAGIC for c in range(8)]
    d_ = [xs[c] - f_[c] for c in range(8)]
    a_ = [jnp.abs(d_[c]) for c in range(8)]
    sgn = [jnp.where(d_[c] < 0.0, -1.0, 1.0) for c in range(8)]

    def tree(op, vs):
        t0 = op(vs[0], vs[1]); t1 = op(vs[2], vs[3])
        t2 = op(vs[4], vs[5]); t3 = op(vs[6], vs[7])
        return op(op(t0, t1), op(t2, t3))

    m_a = tree(jnp.maximum, a_)
    m_n = tree(jnp.minimum, a_)
    sum_f = tree(jnp.add, f_)
    sum_s = tree(jnp.add, sgn)
    sum_a = tree(jnp.add, a_)

    def odd(v):
        u = lax.bitcast_convert_type(v + MAGIC, jnp.int32)
        return (u & 1) == 1

    odd_a = odd(sum_f)
    sum_fb = sum_f - (8.0 - sum_s) * 0.5
    odd_b = odd(sum_fb)

    adj_a = jnp.where(odd_a, 1.0 - 2.0 * m_a, 0.0)
    adj_b = jnp.where(odd_b, 2.0 * m_n, 0.0)
    win_b = ((2.0 - sum_a) + adj_b) - adj_a < 0.0

    m_w = jnp.where(win_b, m_n, m_a)
    k_w = jnp.zeros((16,), jnp.int32)
    s_w = jnp.zeros((16,), jnp.float32)
    for c in range(7, -1, -1):
        ck = a_[c] == m_w
        k_w = jnp.where(ck, c, k_w)
        s_w = jnp.where(ck, sgn[c], s_w)
    fix_w = jnp.where(win_b, -s_w, s_w)
    odd_w = jnp.where(win_b, odd_b, odd_a)
    val_w = jnp.where(odd_w, fix_w, 0.0)

    ys = []
    for c in range(8):
        yc = f_[c] + jnp.where(win_b, 0.5 * sgn[c], 0.0)
        yc = yc + jnp.where(k_w == c, val_w, 0.0)
        ys.append(yc)
    return ys


def _compute_chunk(buf_in, buf_out):
    def group_body(g, c2):
        for u in range(2):
            o = g * 32 + u * 16
            xs = [buf_in[c, pl.ds(o, 16)] for c in range(8)]
            ys = _quantize_group(xs)
            for c in range(8):
                buf_out[c, pl.ds(o, 16)] = ys[c]
        return c2

    lax.fori_loop(0, GROUPS // 2, group_body, 0)


def _sc_body(xt_hbm, out_hbm, in0, in1, ou0, ou1, si0, si1, so0, so1):
    cid = lax.axis_index("c")
    sid = lax.axis_index("s")
    wid = sid * NC + cid
    base_col = wid * COLS_PER_W
    ins, outs = (in0, in1), (ou0, ou1)
    sin, son = (si0, si1), (so0, so1)

    def src(i):
        return xt_hbm.at[:, pl.ds(base_col + i * COLS_CHUNK, COLS_CHUNK)]

    def dst(i):
        return out_hbm.at[:, pl.ds(base_col + i * COLS_CHUNK, COLS_CHUNK)]

    pltpu.async_copy(src(0), ins[0], sin[0])

    def pair_body(j, carry):
        for b in (0, 1):
            i = 2 * j + b
            nb = 1 - b

            @pl.when(i + 1 < N_CHUNKS)
            def _():
                pltpu.async_copy(src(i + 1), ins[nb], sin[nb])

            pltpu.make_async_copy(src(i), ins[b], sin[b]).wait()

            @pl.when(i >= 2)
            def _():
                pltpu.make_async_copy(outs[b], dst(i - 2), son[b]).wait()

            _compute_chunk(ins[b], outs[b])
            pltpu.async_copy(outs[b], dst(i), son[b])
        return carry

    lax.fori_loop(0, N_CHUNKS // 2, pair_body, 0)
    pltpu.make_async_copy(outs[0], dst(N_CHUNKS - 2), son[0]).wait()
    pltpu.make_async_copy(outs[1], dst(N_CHUNKS - 1), son[1]).wait()


@jax.jit
def _e8_quantize_t(xt):
    run = functools.partial(
        pl.kernel,
        out_type=jax.ShapeDtypeStruct((8, N_ROWS), jnp.float32),
        mesh=plsc.VectorSubcoreMesh(core_axis_name="c", subcore_axis_name="s"),
        scratch_types=[
            pltpu.VMEM((8, COLS_CHUNK), jnp.float32),
            pltpu.VMEM((8, COLS_CHUNK), jnp.float32),
            pltpu.VMEM((8, COLS_CHUNK), jnp.float32),
            pltpu.VMEM((8, COLS_CHUNK), jnp.float32),
            pltpu.SemaphoreType.DMA,
            pltpu.SemaphoreType.DMA,
            pltpu.SemaphoreType.DMA,
            pltpu.SemaphoreType.DMA,
        ],
        compiler_params=pltpu.CompilerParams(use_tc_tiling_on_sc=True),
    )
    return run(_sc_body)(xt)


def kernel(x):
    return _e8_quantize_t(x.T).T


# speedup vs baseline: 13.5017x; 1.0562x over previous
import functools

import jax
import jax.numpy as jnp
from jax import lax
from jax.experimental import pallas as pl
from jax.experimental.pallas import tpu as pltpu
from jax.experimental.pallas import tpu_sc as plsc

N_ROWS = 2097152
NC = 2
NS = 16
NW = NC * NS
COLS_PER_W = N_ROWS // NW
COLS_CHUNK = 2048
N_CHUNKS = COLS_PER_W // COLS_CHUNK
GROUPS = COLS_CHUNK // 16

MAGIC = 12582912.0


def _quantize_group(xs):
    f_ = [(xs[c] + MAGIC) - MAGIC for c in range(8)]
    d_ = [xs[c] - f_[c] for c in range(8)]
    a_ = [jnp.abs(d_[c]) for c in range(8)]
    neg = [d_[c] < 0.0 for c in range(8)]
    sgn = [jnp.where(neg[c], -1.0, 1.0) for c in range(8)]

    def tree(op, vs):
        t0 = op(vs[0], vs[1]); t1 = op(vs[2], vs[3])
        t2 = op(vs[4], vs[5]); t3 = op(vs[6], vs[7])
        return op(op(t0, t1), op(t2, t3))

    m_a = tree(jnp.maximum, a_)
    m_n = tree(jnp.minimum, a_)
    sum_f = tree(jnp.add, f_)
    sum_a = tree(jnp.add, a_)

    u_a = lax.bitcast_convert_type(sum_f + MAGIC, jnp.int32)
    odd_a = (u_a & 1) == 1
    odd_b = odd_a ^ tree(jnp.logical_xor, neg)

    adj_a = jnp.where(odd_a, 1.0 - 2.0 * m_a, 0.0)
    adj_b = jnp.where(odd_b, 2.0 * m_n, 0.0)
    win_b = ((2.0 - sum_a) + adj_b) - adj_a < 0.0

    m_w = jnp.where(win_b, m_n, m_a)
    k_w = jnp.zeros((16,), jnp.int32)
    s_w = jnp.zeros((16,), jnp.float32)
    for c in range(7, -1, -1):
        ck = a_[c] == m_w
        k_w = jnp.where(ck, c, k_w)
        s_w = jnp.where(ck, sgn[c], s_w)
    fix_w = jnp.where(win_b, -s_w, s_w)
    odd_w = jnp.where(win_b, odd_b, odd_a)
    val_w = jnp.where(odd_w, fix_w, 0.0)
    halfwin = jnp.where(win_b, 0.5, 0.0)

    ys = []
    for c in range(8):
        yc = f_[c] + halfwin * sgn[c]
        yc = yc + jnp.where(k_w == c, val_w, 0.0)
        ys.append(yc)
    return ys


def _compute_chunk(buf_in, buf_out):
    def group_body(g, c2):
        for u in range(4):
            o = g * 64 + u * 16
            xs = [buf_in[c, pl.ds(o, 16)] for c in range(8)]
            ys = _quantize_group(xs)
            for c in range(8):
                buf_out[c, pl.ds(o, 16)] = ys[c]
        return c2

    lax.fori_loop(0, GROUPS // 4, group_body, 0)


def _sc_body(xt_hbm, out_hbm, in0, in1, ou0, ou1, si0, si1, so0, so1):
    cid = lax.axis_index("c")
    sid = lax.axis_index("s")
    wid = sid * NC + cid
    base_col = wid * COLS_PER_W
    ins, outs = (in0, in1), (ou0, ou1)
    sin, son = (si0, si1), (so0, so1)

    def src(i):
        return xt_hbm.at[:, pl.ds(base_col + i * COLS_CHUNK, COLS_CHUNK)]

    def dst(i):
        return out_hbm.at[:, pl.ds(base_col + i * COLS_CHUNK, COLS_CHUNK)]

    pltpu.async_copy(src(0), ins[0], sin[0])

    def pair_body(j, carry):
        for b in (0, 1):
            i = 2 * j + b
            nb = 1 - b

            @pl.when(i + 1 < N_CHUNKS)
            def _():
                pltpu.async_copy(src(i + 1), ins[nb], sin[nb])

            pltpu.make_async_copy(src(i), ins[b], sin[b]).wait()

            @pl.when(i >= 2)
            def _():
                pltpu.make_async_copy(outs[b], dst(i - 2), son[b]).wait()

            _compute_chunk(ins[b], outs[b])
            pltpu.async_copy(outs[b], dst(i), son[b])
        return carry

    lax.fori_loop(0, N_CHUNKS // 2, pair_body, 0)
    pltpu.make_async_copy(outs[0], dst(N_CHUNKS - 2), son[0]).wait()
    pltpu.make_async_copy(outs[1], dst(N_CHUNKS - 1), son[1]).wait()


@jax.jit
def _e8_quantize_t(xt):
    run = functools.partial(
        pl.kernel,
        out_type=jax.ShapeDtypeStruct((8, N_ROWS), jnp.float32),
        mesh=plsc.VectorSubcoreMesh(core_axis_name="c", subcore_axis_name="s"),
        scratch_types=[
            pltpu.VMEM((8, COLS_CHUNK), jnp.float32),
            pltpu.VMEM((8, COLS_CHUNK), jnp.float32),
            pltpu.VMEM((8, COLS_CHUNK), jnp.float32),
            pltpu.VMEM((8, COLS_CHUNK), jnp.float32),
            pltpu.SemaphoreType.DMA,
            pltpu.SemaphoreType.DMA,
            pltpu.SemaphoreType.DMA,
            pltpu.SemaphoreType.DMA,
        ],
        compiler_params=pltpu.CompilerParams(use_tc_tiling_on_sc=True),
    )
    return run(_sc_body)(xt)


def kernel(x):
    return _e8_quantize_t(x.T).T


# speedup vs baseline: 14.7319x; 1.0911x over previous
import functools

import jax
import jax.numpy as jnp
from jax import lax
from jax.experimental import pallas as pl
from jax.experimental.pallas import tpu as pltpu
from jax.experimental.pallas import tpu_sc as plsc

N_ROWS = 2097152
NC = 2
NS = 16
NW = NC * NS
COLS_PER_W = N_ROWS // NW
COLS_CHUNK = 2048
N_CHUNKS = COLS_PER_W // COLS_CHUNK
GROUPS = COLS_CHUNK // 16

MAGIC = 12582912.0


def _quantize_group(xs):
    f_ = [(xs[c] + MAGIC) - MAGIC for c in range(8)]
    d_ = [xs[c] - f_[c] for c in range(8)]
    a_ = [jnp.abs(d_[c]) for c in range(8)]
    neg = [d_[c] < 0.0 for c in range(8)]
    sgn = [jnp.where(neg[c], -1.0, 1.0) for c in range(8)]

    def tree(op, vs):
        t0 = op(vs[0], vs[1]); t1 = op(vs[2], vs[3])
        t2 = op(vs[4], vs[5]); t3 = op(vs[6], vs[7])
        return op(op(t0, t1), op(t2, t3))

    m_a = tree(jnp.maximum, a_)
    m_n = tree(jnp.minimum, a_)
    sum_f = tree(jnp.add, f_)
    sum_a = tree(jnp.add, a_)

    u_a = lax.bitcast_convert_type(sum_f + MAGIC, jnp.int32)
    odd_a = (u_a & 1) == 1
    odd_b = odd_a ^ tree(jnp.logical_xor, neg)

    adj_a = jnp.where(odd_a, 1.0 - 2.0 * m_a, 0.0)
    adj_b = jnp.where(odd_b, 2.0 * m_n, 0.0)
    win_b = ((2.0 - sum_a) + adj_b) - adj_a < 0.0

    m_w = jnp.where(win_b, m_n, m_a)
    k_w = jnp.zeros((16,), jnp.int32)
    s_w = jnp.zeros((16,), jnp.float32)
    for c in range(7, -1, -1):
        ck = a_[c] == m_w
        k_w = jnp.where(ck, c, k_w)
        s_w = jnp.where(ck, sgn[c], s_w)
    fix_w = jnp.where(win_b, -s_w, s_w)
    odd_w = jnp.where(win_b, odd_b, odd_a)
    val_w = jnp.where(odd_w, fix_w, 0.0)
    halfwin = jnp.where(win_b, 0.5, 0.0)

    ys = [f_[c] + halfwin * sgn[c] for c in range(8)]
    return ys, k_w, val_w


def _compute_chunk(buf_in, buf_out):
    iota = lax.iota(jnp.int32, 16)

    def group_body(g, c2):
        for u in range(4):
            o = g * 64 + u * 16
            xs = [buf_in[c, pl.ds(o, 16)] for c in range(8)]
            ys, k_w, val_w = _quantize_group(xs)
            for c in range(8):
                buf_out[c, pl.ds(o, 16)] = ys[c]
            plsc.addupdate_scatter(buf_out, [k_w, iota + o], val_w)
        return c2

    lax.fori_loop(0, GROUPS // 4, group_body, 0)


def _sc_body(xt_hbm, out_hbm, in0, in1, ou0, ou1, si0, si1, so0, so1):
    cid = lax.axis_index("c")
    sid = lax.axis_index("s")
    wid = sid * NC + cid
    base_col = wid * COLS_PER_W
    ins, outs = (in0, in1), (ou0, ou1)
    sin, son = (si0, si1), (so0, so1)

    def src(i):
        return xt_hbm.at[:, pl.ds(base_col + i * COLS_CHUNK, COLS_CHUNK)]

    def dst(i):
        return out_hbm.at[:, pl.ds(base_col + i * COLS_CHUNK, COLS_CHUNK)]

    pltpu.async_copy(src(0), ins[0], sin[0])

    def pair_body(j, carry):
        for b in (0, 1):
            i = 2 * j + b
            nb = 1 - b

            @pl.when(i + 1 < N_CHUNKS)
            def _():
                pltpu.async_copy(src(i + 1), ins[nb], sin[nb])

            pltpu.make_async_copy(src(i), ins[b], sin[b]).wait()

            @pl.when(i >= 2)
            def _():
                pltpu.make_async_copy(outs[b], dst(i - 2), son[b]).wait()

            _compute_chunk(ins[b], outs[b])
            pltpu.async_copy(outs[b], dst(i), son[b])
        return carry

    lax.fori_loop(0, N_CHUNKS // 2, pair_body, 0)
    pltpu.make_async_copy(outs[0], dst(N_CHUNKS - 2), son[0]).wait()
    pltpu.make_async_copy(outs[1], dst(N_CHUNKS - 1), son[1]).wait()


@jax.jit
def _e8_quantize_t(xt):
    run = functools.partial(
        pl.kernel,
        out_type=jax.ShapeDtypeStruct((8, N_ROWS), jnp.float32),
        mesh=plsc.VectorSubcoreMesh(core_axis_name="c", subcore_axis_name="s"),
        scratch_types=[
            pltpu.VMEM((8, COLS_CHUNK), jnp.float32),
            pltpu.VMEM((8, COLS_CHUNK), jnp.float32),
            pltpu.VMEM((8, COLS_CHUNK), jnp.float32),
            pltpu.VMEM((8, COLS_CHUNK), jnp.float32),
            pltpu.SemaphoreType.DMA,
            pltpu.SemaphoreType.DMA,
            pltpu.SemaphoreType.DMA,
            pltpu.SemaphoreType.DMA,
        ],
        compiler_params=pltpu.CompilerParams(
            needs_layout_passes=False, use_tc_tiling_on_sc=True),
    )
    return run(_sc_body)(xt)


def kernel(x):
    return _e8_quantize_t(x.T).T
